# baseline (speedup 1.0000x reference)
import jax
import jax.numpy as jnp
from jax.experimental import pallas as pl
from jax.experimental.pallas import tpu as pltpu

B, A, P, V, D = 4096, 200, 8, 92, 64
RB = 16
R = RB * A
G = B // RB


def _body(idx_ref, prop_ref, table_ref, w1_ref, b1_ref, w2_ref, b2_ref, out_ref):
    idx = idx_ref[...]
    lanes = jax.lax.broadcasted_iota(jnp.int32, (R, 128), 1)
    onehot = (idx == lanes).astype(jnp.bfloat16)
    elem = jnp.dot(onehot, table_ref[...], preferred_element_type=jnp.float32)
    props = prop_ref[...].reshape(R, P).astype(jnp.bfloat16)
    h = jnp.dot(props, w1_ref[...], preferred_element_type=jnp.float32)
    h = jnp.maximum(h + b1_ref[...], 0.0).astype(jnp.bfloat16)
    prop = jnp.dot(h, w2_ref[...], preferred_element_type=jnp.float32)
    out_ref[...] = (elem + prop + b2_ref[...]).reshape(RB, A, D)


def kernel(atomic_numbers, atomic_properties, table, W1, b1, W2, b2):
    idx = (atomic_numbers.astype(jnp.int32) - 1).reshape(B * A, 1)
    table_pad = jnp.zeros((128, D), jnp.bfloat16).at[:V].set(table.astype(jnp.bfloat16))
    out = pl.pallas_call(
        _body,
        grid=(G,),
        in_specs=[
            pl.BlockSpec((R, 1), lambda i: (i, 0)),
            pl.BlockSpec((RB, A, P), lambda i: (i, 0, 0)),
            pl.BlockSpec((128, D), lambda i: (0, 0)),
            pl.BlockSpec((P, 32), lambda i: (0, 0)),
            pl.BlockSpec((1, 32), lambda i: (0, 0)),
            pl.BlockSpec((32, D), lambda i: (0, 0)),
            pl.BlockSpec((1, D), lambda i: (0, 0)),
        ],
        out_specs=pl.BlockSpec((RB, A, D), lambda i: (i, 0, 0)),
        out_shape=jax.ShapeDtypeStruct((B, A, D), jnp.float32),
        compiler_params=pltpu.CompilerParams(
            dimension_semantics=("arbitrary",),
        ),
    )(idx, atomic_properties, table_pad,
      W1.astype(jnp.bfloat16), b1.reshape(1, 32), W2.astype(jnp.bfloat16),
      b2.reshape(1, D))
    return out


# speedup vs baseline: 5.9533x; 5.9533x over previous
import jax
import jax.numpy as jnp
from jax.experimental import pallas as pl
from jax.experimental.pallas import tpu as pltpu

B, A, P, V, D = 4096, 200, 8, 92, 64
AT = 8
BT = 1024
H = 32


def _body(an_ref, prop_ref, tab_ref, w1bd_ref, b1bd_ref, w2bd_ref, b2_ref,
          out_ref):
    idx = an_ref[...] - 1
    props = prop_ref[...].reshape(AT * P, BT).astype(jnp.bfloat16)
    h = jnp.dot(w1bd_ref[...], props, preferred_element_type=jnp.float32)
    h = jnp.maximum(h + b1bd_ref[...], 0.0).astype(jnp.bfloat16)
    tab = tab_ref[...]
    for g in range(AT // 4):
        prop4 = jnp.dot(w2bd_ref[...], h[g * 4 * H:(g + 1) * 4 * H, :],
                        preferred_element_type=jnp.float32)
        for j in range(4):
            a = g * 4 + j
            idx_b = jnp.broadcast_to(idx[a:a + 1, :], (D, BT))
            elem = jnp.take_along_axis(tab, idx_b, axis=1)
            out_ref[a, :, :] = elem + prop4[j * D:(j + 1) * D, :] + b2_ref[...]


def kernel(atomic_numbers, atomic_properties, table, W1, b1, W2, b2):
    anT = atomic_numbers.astype(jnp.int32).T
    propsT = jnp.transpose(atomic_properties, (1, 2, 0))
    tabT = jnp.zeros((D, 128), jnp.float32).at[:, :V].set(table.T)
    w1bd = jnp.zeros((AT * H, AT * P), jnp.bfloat16)
    w2bd = jnp.zeros((4 * D, 4 * H), jnp.bfloat16)
    w1t = W1.T.astype(jnp.bfloat16)
    w2t = W2.T.astype(jnp.bfloat16)
    for i in range(AT):
        w1bd = w1bd.at[i * H:(i + 1) * H, i * P:(i + 1) * P].set(w1t)
    for i in range(4):
        w2bd = w2bd.at[i * D:(i + 1) * D, i * H:(i + 1) * H].set(w2t)
    b1bd = jnp.tile(b1, AT).reshape(AT * H, 1)
    b2c = b2.reshape(D, 1)

    outT = pl.pallas_call(
        _body,
        grid=(A // AT, B // BT),
        in_specs=[
            pl.BlockSpec((AT, BT), lambda i, j: (i, j)),
            pl.BlockSpec((AT, P, BT), lambda i, j: (i, 0, j)),
            pl.BlockSpec((D, 128), lambda i, j: (0, 0)),
            pl.BlockSpec((AT * H, AT * P), lambda i, j: (0, 0)),
            pl.BlockSpec((AT * H, 1), lambda i, j: (0, 0)),
            pl.BlockSpec((4 * D, 4 * H), lambda i, j: (0, 0)),
            pl.BlockSpec((D, 1), lambda i, j: (0, 0)),
        ],
        out_specs=pl.BlockSpec((AT, D, BT), lambda i, j: (i, 0, j)),
        out_shape=jax.ShapeDtypeStruct((A, D, B), jnp.float32),
        compiler_params=pltpu.CompilerParams(
            dimension_semantics=("arbitrary", "arbitrary"),
        ),
    )(anT, propsT, tabT, w1bd, b1bd, w2bd, b2c)
    return jnp.transpose(outT, (2, 0, 1))


# speedup vs baseline: 6.3775x; 1.0713x over previous
import jax
import jax.numpy as jnp
from jax.experimental import pallas as pl
from jax.experimental.pallas import tpu as pltpu

B, A, P, V, D = 4096, 200, 8, 92, 64
AT = 8
BT = 2048
H = 32


def _body(an_ref, prop_ref, tab_ref, w1bd_ref, b1bd_ref, w2bd_ref, b2_ref,
          out_ref):
    idx = an_ref[...] - 1
    props = prop_ref[...].reshape(AT * P, BT).astype(jnp.bfloat16)
    h = jnp.dot(w1bd_ref[...], props, preferred_element_type=jnp.float32)
    h = jnp.maximum(h + b1bd_ref[...], 0.0).astype(jnp.bfloat16)
    tab = tab_ref[...]
    for g in range(AT // 4):
        prop4 = jnp.dot(w2bd_ref[...], h[g * 4 * H:(g + 1) * 4 * H, :],
                        preferred_element_type=jnp.float32)
        for j in range(4):
            a = g * 4 + j
            idx_b = jnp.broadcast_to(idx[a:a + 1, :], (D, BT))
            elem = jnp.take_along_axis(tab, idx_b, axis=1)
            out_ref[a, :, :] = elem + prop4[j * D:(j + 1) * D, :] + b2_ref[...]


def kernel(atomic_numbers, atomic_properties, table, W1, b1, W2, b2):
    anT = atomic_numbers.astype(jnp.int32).T
    propsT = jnp.transpose(atomic_properties, (1, 2, 0))
    tabT = jnp.zeros((D, 128), jnp.float32).at[:, :V].set(table.T)
    w1bd = jnp.zeros((AT * H, AT * P), jnp.bfloat16)
    w2bd = jnp.zeros((4 * D, 4 * H), jnp.bfloat16)
    w1t = W1.T.astype(jnp.bfloat16)
    w2t = W2.T.astype(jnp.bfloat16)
    for i in range(AT):
        w1bd = w1bd.at[i * H:(i + 1) * H, i * P:(i + 1) * P].set(w1t)
    for i in range(4):
        w2bd = w2bd.at[i * D:(i + 1) * D, i * H:(i + 1) * H].set(w2t)
    b1bd = jnp.tile(b1, AT).reshape(AT * H, 1)
    b2c = b2.reshape(D, 1)

    outT = pl.pallas_call(
        _body,
        grid=(A // AT, B // BT),
        in_specs=[
            pl.BlockSpec((AT, BT), lambda i, j: (i, j)),
            pl.BlockSpec((AT, P, BT), lambda i, j: (i, 0, j)),
            pl.BlockSpec((D, 128), lambda i, j: (0, 0)),
            pl.BlockSpec((AT * H, AT * P), lambda i, j: (0, 0)),
            pl.BlockSpec((AT * H, 1), lambda i, j: (0, 0)),
            pl.BlockSpec((4 * D, 4 * H), lambda i, j: (0, 0)),
            pl.BlockSpec((D, 1), lambda i, j: (0, 0)),
        ],
        out_specs=pl.BlockSpec((AT, D, BT), lambda i, j: (i, 0, j)),
        out_shape=jax.ShapeDtypeStruct((A, D, B), jnp.float32),
        compiler_params=pltpu.CompilerParams(
            dimension_semantics=("arbitrary", "arbitrary"),
        ),
    )(anT, propsT, tabT, w1bd, b1bd, w2bd, b2c)
    return jnp.transpose(outT, (2, 0, 1))
